# baseline (speedup 1.0000x reference)
import functools

import jax
import jax.numpy as jnp
from jax import lax
from jax.experimental import pallas as pl
from jax.experimental.pallas import tpu as pltpu
from jax.experimental.pallas import tpu_sc as plsc

_B, _N, _M, _NB = 64, 128, 256, 10
_AF, _BF, _H = 128, 16, 256
_DEPTH = 3

_NW = 32
_MPW = _B // _NW
_AC = 8
_EC = _AC * _NB
_NCHUNK = _N // _AC
_NCH = _MPW * _N // _AC
_EPW = _MPW * _N * _NB
_PAD = 8

_f32 = jnp.float32


def _dot(a, b):
    return jax.lax.dot_general(a, b, (((1,), (0,)), ((), ())),
                               preferred_element_type=_f32)


def _tc_pre_body(af_ref, bf_ref, w1_ref, wnh_ref, wnb_ref, bn_ref, w2b_ref,
                 h_ref, hn_ref, gbt_ref, tb2_ref):
    h = jnp.maximum(_dot(af_ref[...], w1_ref[...]), 0.0)
    h_ref[...] = h
    hn_ref[...] = _dot(h, wnh_ref[...])
    gbt_ref[...] = _dot(bf_ref[...], wnb_ref[...]) + bn_ref[...]
    tb2_ref[...] = _dot(bf_ref[...], w2b_ref[...])


def _tc_update_body(h_ref, nei_ref, wah_ref, wan_ref, ba_ref, wnh_ref,
                    h2_ref, hn2_ref, last):
    h2 = jnp.maximum(
        _dot(h_ref[...], wah_ref[...]) + _dot(nei_ref[...], wan_ref[...])
        + ba_ref[...], 0.0)
    h2_ref[...] = h2
    hn2_ref[...] = _dot(h2, wnh_ref[...])


def _tc_final_body(h_ref, nei_ref, matom_ref, w2_ref, out_ref):
    out_ref[...] = _dot(h_ref[...], w2_ref[...]) * nei_ref[...] * matom_ref[...]


_ROWS_PER_STEP = 1024


def _rows_spec(cols):
    return pl.BlockSpec((_ROWS_PER_STEP, cols), lambda i: (i, 0))


def _rep_spec(*blk):
    return pl.BlockSpec(blk, lambda i: (0,) * len(blk))


def _tc_pre(af, bf, W1, wnh, wnb, bn2, W2b):
    grid = (_B * _N // _ROWS_PER_STEP,)
    bond_rows = _B * _M // (_B * _N // _ROWS_PER_STEP)
    return pl.pallas_call(
        _tc_pre_body,
        grid=grid,
        in_specs=[
            _rows_spec(_AF),
            pl.BlockSpec((bond_rows, _BF), lambda i: (i, 0)),
            _rep_spec(_AF, _H), _rep_spec(_H, _H), _rep_spec(_BF, _H),
            _rep_spec(1, _H), _rep_spec(_BF, _H),
        ],
        out_specs=[
            _rows_spec(_H), _rows_spec(_H),
            pl.BlockSpec((bond_rows, _H), lambda i: (i, 0)),
            pl.BlockSpec((bond_rows, _H), lambda i: (i, 0)),
        ],
        out_shape=[
            jax.ShapeDtypeStruct((_B * _N, _H), _f32),
            jax.ShapeDtypeStruct((_B * _N, _H), _f32),
            jax.ShapeDtypeStruct((_B * _M, _H), _f32),
            jax.ShapeDtypeStruct((_B * _M, _H), _f32),
        ],
    )(af, bf, W1, wnh, wnb, bn2, W2b)


def _tc_update(h, nei, wah, wan, ba2, wnext):
    grid = (_B * _N // _ROWS_PER_STEP,)
    return pl.pallas_call(
        functools.partial(_tc_update_body, last=False),
        grid=grid,
        in_specs=[
            _rows_spec(_H), _rows_spec(_H),
            _rep_spec(_H, _H), _rep_spec(_H, _H), _rep_spec(1, _H),
            _rep_spec(_H, _H),
        ],
        out_specs=[_rows_spec(_H), _rows_spec(_H)],
        out_shape=[
            jax.ShapeDtypeStruct((_B * _N, _H), _f32),
            jax.ShapeDtypeStruct((_B * _N, _H), _f32),
        ],
    )(h, nei, wah, wan, ba2, wnext)


def _tc_final(h, nei, matom, W2):
    grid = (_B * _N // _ROWS_PER_STEP,)
    return pl.pallas_call(
        _tc_final_body,
        grid=grid,
        in_specs=[
            _rows_spec(_H), _rows_spec(_H),
            pl.BlockSpec((_ROWS_PER_STEP, 1), lambda i: (i, 0)),
            _rep_spec(_H, _H),
        ],
        out_specs=_rows_spec(_H),
        out_shape=jax.ShapeDtypeStruct((_B * _N, _H), _f32),
    )(h, nei, matom, W2)


def _sc_agg_body(product, atab_hbm, btab_hbm, ia_hbm, ib_hbm, out_hbm,
                 ia_v, ib_v, ga0, gb0, ga1, gb1, nei0, nei1,
                 sa0, sb0, sa1, sb1):
    wid = lax.axis_index("s") * 2 + lax.axis_index("c")
    ebase = wid * _EPW
    abase = wid * (_MPW * _N)
    pltpu.sync_copy(ia_hbm.at[pl.ds(ebase, _EPW)], ia_v)
    pltpu.sync_copy(ib_hbm.at[pl.ds(ebase, _EPW)], ib_v)

    bufs = ((ga0, gb0, nei0, sa0, sb0), (ga1, gb1, nei1, sa1, sb1))

    def start(c, p):
        ga, gb, _, sa, sb = bufs[p]
        e0 = c * _EC
        pltpu.make_async_copy(
            atab_hbm.at[ia_v.at[pl.ds(e0, _EC)]], ga, sa).start()
        pltpu.make_async_copy(
            btab_hbm.at[ib_v.at[pl.ds(e0, _EC)]], gb, sb).start()

    def wait(c, p):
        ga, gb, _, sa, sb = bufs[p]
        e0 = c * _EC
        pltpu.make_async_copy(
            atab_hbm.at[ia_v.at[pl.ds(e0, _EC)]], ga, sa).wait()
        pltpu.make_async_copy(
            btab_hbm.at[ib_v.at[pl.ds(e0, _EC)]], gb, sb).wait()

    start(0, 0)
    start(1, 1)

    def pair_body(cc, _):
        c0 = cc * 2
        for p in range(2):
            c = c0 + p
            ga, gb, nei, _, _ = bufs[p]
            wait(c, p)

            def chan_body(k, _):
                s = pl.ds(k * 16, 16)
                for a in range(_AC):
                    acc = None
                    for nb in range(_NB):
                        e = a * _NB + nb
                        if product:
                            t = ga[e, s] * gb[e, s]
                        else:
                            t = jnp.maximum(ga[e, s] + gb[e, s], 0.0)
                        acc = t if acc is None else acc + t
                    nei[a, s] = acc
                return 0

            lax.fori_loop(0, _H // 16, chan_body, 0)
            pltpu.sync_copy(nei, out_hbm.at[pl.ds(abase + c * _AC, _AC)])

            @pl.when(c + 2 < _NCH)
            def _():
                start(c + 2, p)
        return 0

    lax.fori_loop(0, _NCH // 2, pair_body, 0)


def _sc_agg(atab, btab, ia, ib, product):
    mesh = plsc.VectorSubcoreMesh(core_axis_name="c", subcore_axis_name="s")
    kfn = functools.partial(
        pl.kernel,
        mesh=mesh,
        out_type=jax.ShapeDtypeStruct((_B * _N, _H), _f32),
        scratch_types=[
            pltpu.VMEM((_EPW,), jnp.int32),
            pltpu.VMEM((_EPW,), jnp.int32),
            pltpu.VMEM((_EC, _H), _f32),
            pltpu.VMEM((_EC, _H), _f32),
            pltpu.VMEM((_EC, _H), _f32),
            pltpu.VMEM((_EC, _H), _f32),
            pltpu.VMEM((_AC, _H), _f32),
            pltpu.VMEM((_AC, _H), _f32),
            pltpu.SemaphoreType.DMA,
            pltpu.SemaphoreType.DMA,
            pltpu.SemaphoreType.DMA,
            pltpu.SemaphoreType.DMA,
        ],
    )(functools.partial(_sc_agg_body, product))
    return kfn(atab, btab, ia, ib)


@jax.jit
def kernel(atom_feats, bond_feats, atom_graph, bond_graph, num_nbs, n_atoms,
           mask_neis, mask_atoms, W1, Wn, bn, Wa, ba, W2a, W2b, W2):
    del num_nbs, n_atoms
    boff = (jnp.arange(_B, dtype=jnp.int32) * _N)[:, None, None]
    ia = (atom_graph.astype(jnp.int32) + boff).reshape(-1)
    mflat = mask_neis.reshape(_B, _N, _NB)
    boffm = (jnp.arange(_B, dtype=jnp.int32) * _M)[:, None, None]
    ib = jnp.where(mflat, bond_graph.astype(jnp.int32) + boffm,
                   jnp.int32(_B * _M)).reshape(-1)
    matom = mask_atoms.astype(_f32).reshape(_B * _N, 1)

    wnh, wnb = Wn[:_H], Wn[_H:]
    wah, wan = Wa[:_H], Wa[_H:]
    bn2 = bn.reshape(1, _H)
    ba2 = ba.reshape(1, _H)

    af = atom_feats.reshape(_B * _N, _AF)
    bf = bond_feats.reshape(_B * _M, _BF)

    h, hn, gbt, tb2 = _tc_pre(af, bf, W1, wnh, wnb, bn2, W2b)
    gbt = jnp.concatenate(
        [gbt, jnp.full((_PAD, _H), -1e30, _f32)], axis=0)
    tb2 = jnp.concatenate(
        [tb2, jnp.zeros((_PAD, _H), _f32)], axis=0)

    for _ in range(_DEPTH - 2):
        nei = _sc_agg(hn, gbt, ia, ib, product=False)
        h, hn = _tc_update(h, nei, wah, wan, ba2, wnh)
    nei = _sc_agg(hn, gbt, ia, ib, product=False)
    h, ha = _tc_update(h, nei, wah, wan, ba2, W2a)
    nei = _sc_agg(ha, tb2, ia, ib, product=True)
    local = _tc_final(h, nei, matom, W2)
    return local.reshape(_B, _N, _H)


# speedup vs baseline: 1.8972x; 1.8972x over previous
import functools

import jax
import jax.numpy as jnp
from jax import lax
from jax.experimental import pallas as pl
from jax.experimental.pallas import tpu as pltpu
from jax.experimental.pallas import tpu_sc as plsc

_B, _N, _M, _NB = 64, 128, 256, 10
_AF, _BF, _H = 128, 16, 256
_DEPTH = 3

_NW = 32
_MPW = _B // _NW
_L = 16
_NG = _N // _L
_PAD = 8

_f32 = jnp.float32


def _dot(a, b):
    return jax.lax.dot_general(a, b, (((1,), (0,)), ((), ())),
                               preferred_element_type=_f32)


def _tc_pre_body(af_ref, bf_ref, w1_ref, wnh_ref, wnb_ref, bn_ref, w2b_ref,
                 h_ref, hn_ref, gbt_ref, tb2_ref):
    h = jnp.maximum(_dot(af_ref[...], w1_ref[...]), 0.0)
    h_ref[...] = h
    hn_ref[...] = _dot(h, wnh_ref[...])
    gbt_ref[...] = _dot(bf_ref[...], wnb_ref[...]) + bn_ref[...]
    tb2_ref[...] = _dot(bf_ref[...], w2b_ref[...])


def _tc_update_body(h_ref, nei_ref, wah_ref, wan_ref, ba_ref, wnh_ref,
                    h2_ref, hn2_ref):
    h2 = jnp.maximum(
        _dot(h_ref[...], wah_ref[...]) + _dot(nei_ref[...], wan_ref[...])
        + ba_ref[...], 0.0)
    h2_ref[...] = h2
    hn2_ref[...] = _dot(h2, wnh_ref[...])


def _tc_final_body(h_ref, nei_ref, matom_ref, w2_ref, out_ref):
    out_ref[...] = _dot(h_ref[...], w2_ref[...]) * nei_ref[...] * matom_ref[...]


_ROWS_PER_STEP = 1024


def _rows_spec(cols):
    return pl.BlockSpec((_ROWS_PER_STEP, cols), lambda i: (i, 0))


def _rep_spec(*blk):
    return pl.BlockSpec(blk, lambda i: (0,) * len(blk))


def _tc_pre(af, bf, W1, wnh, wnb, bn2, W2b):
    grid = (_B * _N // _ROWS_PER_STEP,)
    bond_rows = _B * _M // grid[0]
    return pl.pallas_call(
        _tc_pre_body,
        grid=grid,
        in_specs=[
            _rows_spec(_AF),
            pl.BlockSpec((bond_rows, _BF), lambda i: (i, 0)),
            _rep_spec(_AF, _H), _rep_spec(_H, _H), _rep_spec(_BF, _H),
            _rep_spec(1, _H), _rep_spec(_BF, _H),
        ],
        out_specs=[
            _rows_spec(_H), _rows_spec(_H),
            pl.BlockSpec((bond_rows, _H), lambda i: (i, 0)),
            pl.BlockSpec((bond_rows, _H), lambda i: (i, 0)),
        ],
        out_shape=[
            jax.ShapeDtypeStruct((_B * _N, _H), _f32),
            jax.ShapeDtypeStruct((_B * _N, _H), _f32),
            jax.ShapeDtypeStruct((_B * _M, _H), _f32),
            jax.ShapeDtypeStruct((_B * _M, _H), _f32),
        ],
    )(af, bf, W1, wnh, wnb, bn2, W2b)


def _tc_update(h, nei, wah, wan, ba2, wnext):
    grid = (_B * _N // _ROWS_PER_STEP,)
    return pl.pallas_call(
        _tc_update_body,
        grid=grid,
        in_specs=[
            _rows_spec(_H), _rows_spec(_H),
            _rep_spec(_H, _H), _rep_spec(_H, _H), _rep_spec(1, _H),
            _rep_spec(_H, _H),
        ],
        out_specs=[_rows_spec(_H), _rows_spec(_H)],
        out_shape=[
            jax.ShapeDtypeStruct((_B * _N, _H), _f32),
            jax.ShapeDtypeStruct((_B * _N, _H), _f32),
        ],
    )(h, nei, wah, wan, ba2, wnext)


def _tc_final(h, nei, matom, W2):
    grid = (_B * _N // _ROWS_PER_STEP,)
    return pl.pallas_call(
        _tc_final_body,
        grid=grid,
        in_specs=[
            _rows_spec(_H), _rows_spec(_H),
            pl.BlockSpec((_ROWS_PER_STEP, 1), lambda i: (i, 0)),
            _rep_spec(_H, _H),
        ],
        out_specs=_rows_spec(_H),
        out_shape=jax.ShapeDtypeStruct((_B * _N, _H), _f32),
    )(h, nei, matom, W2)


def _sc_agg_body(product, atab_hbm, btab_hbm, iar_hbm, ibr_hbm, out_hbm,
                 at_v, bt_v, ia_v, ib_v, nei_v):
    wid = lax.axis_index("s") * 2 + lax.axis_index("c")
    i32 = jnp.int32

    def mol_body(m, _):
        b = wid * _MPW + m
        pltpu.sync_copy(atab_hbm.at[pl.ds(b * _N * _H, _N * _H)], at_v)
        pltpu.sync_copy(btab_hbm.at[pl.ds(b * _M * _H, _M * _H)],
                        bt_v.at[pl.ds(0, _M * _H)])
        pltpu.sync_copy(btab_hbm.at[pl.ds(_B * _M * _H, _PAD * _H)],
                        bt_v.at[pl.ds(_M * _H, _PAD * _H)])
        pltpu.sync_copy(iar_hbm.at[b], ia_v)
        pltpu.sync_copy(ibr_hbm.at[b], ib_v)

        rowbase = lax.iota(i32, _L) * _H
        for g in range(_NG):
            ra = [ia_v[pl.ds((g * _NB + nb) * _L, _L)] * _H
                  for nb in range(_NB)]
            rb = [ib_v[pl.ds((g * _NB + nb) * _L, _L)] * _H
                  for nb in range(_NB)]

            def chan_body(c, _, ra=ra, rb=rb):
                colv = jnp.full((_L,), c, i32)
                acc = None
                for nb in range(_NB):
                    va = plsc.load_gather(at_v, [ra[nb] + colv])
                    vb = plsc.load_gather(bt_v, [rb[nb] + colv])
                    if product:
                        t = va * vb
                    else:
                        t = jnp.maximum(va + vb, 0.0)
                    acc = t if acc is None else acc + t
                plsc.store_scatter(nei_v, [rowbase + colv], acc)
                return 0

            lax.fori_loop(0, _H, chan_body, 0)
            pltpu.sync_copy(nei_v,
                            out_hbm.at[pl.ds((b * _N + g * _L) * _H,
                                             _L * _H)])
        return 0

    lax.fori_loop(0, _MPW, mol_body, 0)


def _sc_agg(atab, btab, iar, ibr, product):
    mesh = plsc.VectorSubcoreMesh(core_axis_name="c", subcore_axis_name="s")
    kfn = functools.partial(
        pl.kernel,
        mesh=mesh,
        compiler_params=pltpu.CompilerParams(needs_layout_passes=False),
        out_type=jax.ShapeDtypeStruct((_B * _N * _H,), _f32),
        scratch_types=[
            pltpu.VMEM((_N * _H,), _f32),
            pltpu.VMEM(((_M + _PAD) * _H,), _f32),
            pltpu.VMEM((_N * _NB,), jnp.int32),
            pltpu.VMEM((_N * _NB,), jnp.int32),
            pltpu.VMEM((_L * _H,), _f32),
        ],
    )(functools.partial(_sc_agg_body, product))
    return kfn(atab.reshape(-1), btab.reshape(-1), iar, ibr)


@jax.jit
def kernel(atom_feats, bond_feats, atom_graph, bond_graph, num_nbs, n_atoms,
           mask_neis, mask_atoms, W1, Wn, bn, Wa, ba, W2a, W2b, W2):
    del num_nbs, n_atoms
    i32 = jnp.int32
    def reorder(x):
        return (x.reshape(_B, _NG, _L, _NB).transpose(0, 1, 3, 2)
                .reshape(_B, _N * _NB))
    iar = reorder(atom_graph.astype(i32))
    mflat = mask_neis.reshape(_B, _N, _NB)
    ibr = reorder(jnp.where(mflat, bond_graph.astype(i32), i32(_M)))
    matom = mask_atoms.astype(_f32).reshape(_B * _N, 1)

    wnh, wnb = Wn[:_H], Wn[_H:]
    wah, wan = Wa[:_H], Wa[_H:]
    bn2 = bn.reshape(1, _H)
    ba2 = ba.reshape(1, _H)

    af = atom_feats.reshape(_B * _N, _AF)
    bf = bond_feats.reshape(_B * _M, _BF)

    h, hn, gbt, tb2 = _tc_pre(af, bf, W1, wnh, wnb, bn2, W2b)
    gbt = jnp.concatenate([gbt, jnp.full((_PAD, _H), -1e30, _f32)], axis=0)
    tb2 = jnp.concatenate([tb2, jnp.zeros((_PAD, _H), _f32)], axis=0)

    for _ in range(_DEPTH - 2):
        nei = _sc_agg(hn, gbt, iar, ibr, product=False).reshape(_B * _N, _H)
        h, hn = _tc_update(h, nei, wah, wan, ba2, wnh)
    nei = _sc_agg(hn, gbt, iar, ibr, product=False).reshape(_B * _N, _H)
    h, ha = _tc_update(h, nei, wah, wan, ba2, W2a)
    nei = _sc_agg(ha, tb2, iar, ibr, product=True).reshape(_B * _N, _H)
    local = _tc_final(h, nei, matom, W2)
    return local.reshape(_B, _N, _H)


# speedup vs baseline: 16.4410x; 8.6661x over previous
import jax
import jax.numpy as jnp
from jax.experimental import pallas as pl
from jax.experimental.pallas import tpu as pltpu

_B, _N, _M, _NB = 64, 128, 256, 10
_AF, _BF, _H = 128, 16, 256
_DEPTH = 3
_MOL = 4


def _wlnet_body(af_ref, bf_ref, ag_ref, bg_ref, mnei_ref, matom_ref,
                w1_ref, wnh_ref, wnb_ref, bn_ref, wah_ref, wan_ref, ba_ref,
                w2a_ref, w2b_ref, w2_ref, out_ref):
    f32 = jnp.float32

    dot = lambda a, b: jax.lax.dot_general(
        a, b, (((1,), (0,)), ((), ())), preferred_element_type=f32)

    def nbsum(x):
        acc = x[0:_N]
        for k in range(1, _NB):
            acc = acc + x[k * _N:(k + 1) * _N]
        return acc

    w1 = w1_ref[...]
    wnh = wnh_ref[...]
    wnb = wnb_ref[...]
    wah = wah_ref[...]
    wan = wan_ref[...]
    w2a = w2a_ref[...]
    w2b = w2b_ref[...]
    w2 = w2_ref[...]
    bn = bn_ref[...]
    ba = ba_ref[...]

    for m in range(_MOL):
        af = af_ref[m]
        bf = bf_ref[m]
        ag = ag_ref[m]
        bg = bg_ref[m]
        mnei = mnei_ref[m]
        matom = matom_ref[m]

        oha = (jax.lax.broadcasted_iota(jnp.int32, (_NB * _N, _N), 1)
               == ag).astype(f32)
        ohb = (jax.lax.broadcasted_iota(jnp.int32, (_NB * _N, _M), 1)
               == bg).astype(f32)

        h = jnp.maximum(dot(af, w1), 0.0)
        bnei = dot(ohb, bf) * mnei
        gb = dot(bnei, wnb) + bn + (mnei - 1.0) * 1e30
        for _ in range(_DEPTH - 1):
            ga = dot(oha, dot(h, wnh))
            nei = nbsum(jnp.maximum(ga + gb, 0.0))
            h = jnp.maximum(dot(h, wah) + dot(nei, wan) + ba, 0.0)

        a = dot(oha, dot(h, w2a))
        b2 = dot(bnei, w2b)
        nei = nbsum(a * b2)
        out_ref[m] = dot(h, w2) * nei * matom


@jax.jit
def kernel(atom_feats, bond_feats, atom_graph, bond_graph, num_nbs, n_atoms,
           mask_neis, mask_atoms, W1, Wn, bn, Wa, ba, W2a, W2b, W2):
    del num_nbs, n_atoms
    ag = atom_graph.astype(jnp.int32).transpose(0, 2, 1).reshape(_B, _NB * _N, 1)
    bg = bond_graph.astype(jnp.int32).transpose(0, 2, 1).reshape(_B, _NB * _N, 1)
    mnei = mask_neis.astype(jnp.float32).reshape(_B, _N, _NB).transpose(0, 2, 1)
    mnei = mnei.reshape(_B, _NB * _N, 1)
    matom = mask_atoms.astype(jnp.float32)

    wnh, wnb = Wn[:_H], Wn[_H:]
    wah, wan = Wa[:_H], Wa[_H:]
    bn2 = bn.reshape(1, _H)
    ba2 = ba.reshape(1, _H)

    mol = lambda *blk: pl.BlockSpec((_MOL,) + blk,
                                    lambda b: (b,) + (0,) * len(blk))
    rep = lambda *blk: pl.BlockSpec(blk, lambda b: (0,) * len(blk))

    return pl.pallas_call(
        _wlnet_body,
        grid=(_B // _MOL,),
        in_specs=[
            mol(_N, _AF),
            mol(_M, _BF),
            mol(_NB * _N, 1),
            mol(_NB * _N, 1),
            mol(_NB * _N, 1),
            mol(_N, 1),
            rep(_AF, _H),
            rep(_H, _H),
            rep(_BF, _H),
            rep(1, _H),
            rep(_H, _H),
            rep(_H, _H),
            rep(1, _H),
            rep(_H, _H),
            rep(_BF, _H),
            rep(_H, _H),
        ],
        out_specs=mol(_N, _H),
        out_shape=jax.ShapeDtypeStruct((_B, _N, _H), jnp.float32),
        compiler_params=pltpu.CompilerParams(
            dimension_semantics=("arbitrary",),
        ),
    )(atom_feats, bond_feats, ag, bg, mnei, matom,
      W1, wnh, wnb, bn2, wah, wan, ba2, W2a, W2b, W2)
